# baseline (speedup 1.0000x reference)
import functools

import jax
import jax.numpy as jnp
from jax import lax
from jax.experimental import pallas as pl
from jax.experimental.pallas import tpu as pltpu
from jax.experimental.pallas import tpu_sc as plsc

N = 10000
E = 160000
ET = 170000
EPAD = 172032
DIN = 256
DE = 16
H0 = 4
HID = 256
D0 = 1024
EMB = 128
NCH0 = 8
BN_EPS = 1e-5

NB = 10
BNODE = 1000
EB = 128
WATT = 32
ABATCH = EPAD // (WATT * EB)
EBA = 64
ABATCHA = EPAD // (WATT * EBA)
FLA = EPAD * 16 // 128
MBATCH = EPAD // (16 * EB)
NROWP = 10240
STRIPE = 640


def _edge_proj(edge_attr, UEp):
    def body(ea_ref, ue_ref, ale_ref, es_ref):
        i = pl.program_id(0)
        blk = ea_ref[...]
        ale_ref[...] = jnp.dot(blk, ue_ref[...],
                               preferred_element_type=jnp.float32)

        @pl.when(i == 0)
        def _():
            es_ref[...] = jnp.zeros_like(es_ref)
        es_ref[...] += jnp.broadcast_to(
            jnp.sum(blk, axis=0, keepdims=True), es_ref.shape)

    return pl.pallas_call(
        body,
        grid=(20,),
        in_specs=[
            pl.BlockSpec((8000, DE), lambda i: (i, 0)),
            pl.BlockSpec((DE, 16), lambda i: (0, 0)),
        ],
        out_specs=[
            pl.BlockSpec((8000, 16), lambda i: (i, 0)),
            pl.BlockSpec((8, DE), lambda i: (0, 0)),
        ],
        out_shape=[
            jax.ShapeDtypeStruct((E, 16), jnp.float32),
            jax.ShapeDtypeStruct((8, DE), jnp.float32),
        ],
    )(edge_attr, UEp)


def _layer0_mm(x, W0, VS0):
    def body(x_ref, w_ref, vs_ref, h_ref, al_ref):
        c = pl.program_id(1)
        xb = x_ref[...]
        h_ref[0] = jnp.dot(xb, w_ref[...], preferred_element_type=jnp.float32)

        @pl.when(c == 0)
        def _():
            al_ref[...] = jnp.dot(xb, vs_ref[...],
                                  preferred_element_type=jnp.float32)

    return pl.pallas_call(
        body,
        grid=(NB, NCH0),
        in_specs=[
            pl.BlockSpec((BNODE, DIN), lambda i, c: (i, 0)),
            pl.BlockSpec((DIN, 128), lambda i, c: (0, c)),
            pl.BlockSpec((DIN, 8), lambda i, c: (0, 0)),
        ],
        out_specs=[
            pl.BlockSpec((1, BNODE, 128), lambda i, c: (c, i, 0)),
            pl.BlockSpec((BNODE, 8), lambda i, c: (i, 0)),
        ],
        out_shape=[
            jax.ShapeDtypeStruct((NCH0, N, 128), jnp.float32),
            jax.ShapeDtypeStruct((N, 8), jnp.float32),
        ],
    )(x, W0, VS0)


def _bn_stats(h0c):
    def body(h_ref, st_ref):
        i = pl.program_id(1)

        @pl.when(i == 0)
        def _():
            st_ref[...] = jnp.zeros_like(st_ref)
        xb = h_ref[0]
        st_ref[:, 0:1, :] += jnp.sum(xb, axis=0, keepdims=True)[None]
        st_ref[:, 1:2, :] += jnp.sum(xb * xb, axis=0, keepdims=True)[None]

    return pl.pallas_call(
        body,
        grid=(NCH0, NB),
        in_specs=[pl.BlockSpec((1, BNODE, 128), lambda c, i: (c, i, 0))],
        out_specs=pl.BlockSpec((1, 2, 128), lambda c, i: (c, 0, 0)),
        out_shape=jax.ShapeDtypeStruct((NCH0, 2, 128), jnp.float32),
    )(h0c)


def _layer1_mm(h0c, stats, W1r, VS1r):
    def body(h_ref, st_ref, w_ref, vs_ref, h1_ref, al_ref):
        k = pl.program_id(1)
        m = st_ref[0, 0:1, :] / N
        ex2 = st_ref[0, 1:2, :] / N
        inv = 1.0 / jnp.sqrt(ex2 - m * m + BN_EPS)
        xn = (h_ref[0] - m) * inv
        he = jnp.where(xn > 0, xn, jnp.exp(xn) - 1.0)

        @pl.when(k == 0)
        def _():
            h1_ref[...] = jnp.zeros_like(h1_ref)
            al_ref[...] = jnp.zeros_like(al_ref)
        h1_ref[...] += jnp.dot(he, w_ref[0],
                               preferred_element_type=jnp.float32)
        al_ref[...] += jnp.dot(he, vs_ref[0],
                               preferred_element_type=jnp.float32)

    return pl.pallas_call(
        body,
        grid=(NB, NCH0),
        in_specs=[
            pl.BlockSpec((1, BNODE, 128), lambda i, k: (k, i, 0)),
            pl.BlockSpec((1, 2, 128), lambda i, k: (k, 0, 0)),
            pl.BlockSpec((1, 128, EMB), lambda i, k: (k, 0, 0)),
            pl.BlockSpec((1, 128, 8), lambda i, k: (k, 0, 0)),
        ],
        out_specs=[
            pl.BlockSpec((BNODE, EMB), lambda i, k: (i, 0)),
            pl.BlockSpec((BNODE, 8), lambda i, k: (i, 0)),
        ],
        out_shape=[
            jax.ShapeDtypeStruct((N, EMB), jnp.float32),
            jax.ShapeDtypeStruct((N, 8), jnp.float32),
        ],
    )(h0c, stats, W1r, VS1r)


def _decoder_a(p0, p1, b1, L1, bL1):
    def body(x_ref, y_ref, b1_ref, l1_ref, bl1_ref, emb_ref, z1_ref, st_ref):
        i = pl.program_id(0)
        embb = x_ref[...] + y_ref[...] + b1_ref[...]
        emb_ref[...] = embb
        z = jnp.dot(embb, l1_ref[...], preferred_element_type=jnp.float32)
        z = jnp.maximum(z + bl1_ref[...], 0.0)
        z1_ref[...] = z

        @pl.when(i == 0)
        def _():
            st_ref[...] = jnp.zeros_like(st_ref)
        st_ref[:, 0:1, :] += jnp.sum(z, axis=0, keepdims=True)[None]
        st_ref[:, 1:2, :] += jnp.sum(z * z, axis=0, keepdims=True)[None]

    return pl.pallas_call(
        body,
        grid=(NB,),
        in_specs=[
            pl.BlockSpec((BNODE, EMB), lambda i: (i, 0)),
            pl.BlockSpec((BNODE, EMB), lambda i: (i, 0)),
            pl.BlockSpec((1, EMB), lambda i: (0, 0)),
            pl.BlockSpec((EMB, 2 * HID), lambda i: (0, 0)),
            pl.BlockSpec((1, 2 * HID), lambda i: (0, 0)),
        ],
        out_specs=[
            pl.BlockSpec((BNODE, EMB), lambda i: (i, 0)),
            pl.BlockSpec((BNODE, 2 * HID), lambda i: (i, 0)),
            pl.BlockSpec((1, 2, 2 * HID), lambda i: (0, 0, 0)),
        ],
        out_shape=[
            jax.ShapeDtypeStruct((N, EMB), jnp.float32),
            jax.ShapeDtypeStruct((N, 2 * HID), jnp.float32),
            jax.ShapeDtypeStruct((1, 2, 2 * HID), jnp.float32),
        ],
    )(p0, p1, b1, L1, bL1)


def _decoder_b(z1, st1, L2, bL2, L3, bL3):
    def body(z_ref, st_ref, l2_ref, bl2_ref, l3_ref, bl3_ref, out_ref):
        m = st_ref[0, 0:1, :] / N
        ex2 = st_ref[0, 1:2, :] / N
        inv = 1.0 / jnp.sqrt(ex2 - m * m + BN_EPS)
        zn = (z_ref[...] - m) * inv
        z2 = jnp.dot(zn, l2_ref[...], preferred_element_type=jnp.float32)
        z2 = jnp.maximum(z2 + bl2_ref[...], 0.0)
        r = jnp.dot(z2, l3_ref[...], preferred_element_type=jnp.float32)
        out_ref[...] = r + bl3_ref[...]

    return pl.pallas_call(
        body,
        grid=(NB,),
        in_specs=[
            pl.BlockSpec((BNODE, 2 * HID), lambda i: (i, 0)),
            pl.BlockSpec((1, 2, 2 * HID), lambda i: (0, 0, 0)),
            pl.BlockSpec((2 * HID, HID), lambda i: (0, 0)),
            pl.BlockSpec((1, HID), lambda i: (0, 0)),
            pl.BlockSpec((HID, DIN), lambda i: (0, 0)),
            pl.BlockSpec((1, DIN), lambda i: (0, 0)),
        ],
        out_specs=pl.BlockSpec((BNODE, DIN), lambda i: (i, 0)),
        out_shape=jax.ShapeDtypeStruct((N, DIN), jnp.float32),
    )(z1, st1, L2, bL2, L3, bL3)


def _exp_flat(al):
    def body(al_ref, a_ref):
        a_ref[...] = jnp.exp(al_ref[...])

    return pl.pallas_call(
        body,
        grid=(8,),
        in_specs=[pl.BlockSpec((FLA // 8, 128), lambda i: (i, 0))],
        out_specs=pl.BlockSpec((FLA // 8, 128), lambda i: (i, 0)),
        out_shape=jax.ShapeDtypeStruct((FLA, 128), jnp.float32),
    )(al)


@functools.lru_cache(maxsize=None)
def _mesh():
    return plsc.VectorSubcoreMesh(core_axis_name="c", subcore_axis_name="s")


def _zero_acc(acc, zbuf, tid):
    nk = zbuf.shape[1] // 16

    def zrow(r, _):
        for kk in range(nk):
            zbuf[r, pl.ds(kk * 16, 16)] = jnp.zeros((16,), jnp.float32)
        return 0
    lax.fori_loop(0, 128, zrow, 0)
    for i in range(STRIPE // 128):
        base = pl.multiple_of(tid * STRIPE + i * 128, 128)
        pltpu.sync_copy(zbuf, acc.at[pl.ds(base, 128)])


def _writeback(acc, out_hbm, tid):
    base = pl.multiple_of(tid * STRIPE, 8)

    @pl.when(tid < 15)
    def _():
        pltpu.sync_copy(acc.at[pl.ds(base, STRIPE)],
                        out_hbm.at[pl.ds(base, STRIPE)])

    @pl.when(tid == 15)
    def _():
        pltpu.sync_copy(acc.at[pl.ds(base, N - 15 * STRIPE)],
                        out_hbm.at[pl.ds(base, N - 15 * STRIPE)])


@functools.lru_cache(maxsize=None)
def _make_attn_logits(h):
    @functools.partial(
        pl.kernel,
        out_type=jax.ShapeDtypeStruct((FLA, 128), jnp.float32),
        mesh=_mesh(),
        scratch_types=[
            pltpu.VMEM((ABATCHA, EBA), jnp.int32),
            pltpu.VMEM((ABATCHA, EBA), jnp.int32),
            pltpu.VMEM((EBA, 128), jnp.float32),
            pltpu.VMEM((EBA, 128), jnp.float32),
            pltpu.VMEM((8, 128), jnp.float32),
            pltpu.VMEM((8, 128), jnp.float32),
        ],
    )
    def k(srcA_hbm, dstA_hbm, tabS_hbm, tabD_hbm, ale_hbm, al_hbm,
          src_v, dst_v, als_r, ald_r, ale_b, a_b):
        cid = lax.axis_index("c")
        tid = lax.axis_index("s")
        w = cid * 16 + tid

        pltpu.sync_copy(srcA_hbm.at[w], src_v)
        pltpu.sync_copy(dstA_hbm.at[w], dst_v)

        def batch(j, _):
            pltpu.sync_copy(tabS_hbm.at[src_v.at[j]], als_r)
            pltpu.sync_copy(tabD_hbm.at[dst_v.at[j]], ald_r)
            fbase = pl.multiple_of((w * ABATCHA + j) * 8, 8)
            pltpu.sync_copy(ale_hbm.at[pl.ds(fbase, 8)], ale_b)

            def row(r, _):
                fr = r // 8
                fc = (r % 8) * 16
                al = (als_r[r, pl.ds(0, 16)] + ald_r[r, pl.ds(0, 16)]
                      + ale_b[fr, pl.ds(fc, 16)])
                al = jnp.where(al > 0, al, al * jnp.float32(0.2))
                valid = (w * ABATCHA + j) * EBA + r < ET

                @pl.when(valid)
                def _():
                    a_b[fr, pl.ds(fc, 16)] = al

                @pl.when(jnp.logical_not(valid))
                def _():
                    a_b[fr, pl.ds(fc, 16)] = jnp.full(
                        (16,), -1e30, jnp.float32)
                return 0
            lax.fori_loop(0, EBA, row, 0)
            pltpu.sync_copy(a_b, al_hbm.at[pl.ds(fbase, 8)])
            return 0

        lax.fori_loop(0, ABATCHA, batch, 0)

    del h
    return k


@functools.lru_cache(maxsize=None)
def _make_attn_s():
    @functools.partial(
        pl.kernel,
        out_type=[
            jax.ShapeDtypeStruct((N, 128), jnp.float32),
            jax.ShapeDtypeStruct((N, 128), jnp.float32),
        ],
        mesh=_mesh(),
        scratch_types=[
            pltpu.VMEM((ABATCHA, EBA), jnp.int32),
            pltpu.VMEM((8, 128), jnp.float32),
            pltpu.VMEM((EBA, 128), jnp.float32),
            pltpu.VMEM_SHARED((NROWP, 128), jnp.float32),
        ],
    )
    def k(dstA_hbm, a_hbm, sp0_hbm, sp1_hbm, dst_v, a_b, rowbuf, s_acc):
        cid = lax.axis_index("c")
        tid = lax.axis_index("s")
        w = cid * 16 + tid

        pltpu.sync_copy(dstA_hbm.at[w], dst_v)

        def zrow(r, _):
            for kk in range(8):
                rowbuf[r, pl.ds(kk * 16, 16)] = jnp.zeros((16,), jnp.float32)
            return 0
        lax.fori_loop(0, EBA, zrow, 0)
        for i in range(STRIPE // EBA):
            base = pl.multiple_of(tid * STRIPE + i * EBA, 8)
            pltpu.sync_copy(rowbuf, s_acc.at[pl.ds(base, EBA)])
        plsc.subcore_barrier()

        def batch(j, _):
            fbase = pl.multiple_of((w * ABATCHA + j) * 8, 8)
            pltpu.sync_copy(a_hbm.at[pl.ds(fbase, 8)], a_b)

            def row(r, _):
                fr = r // 8
                fc = (r % 8) * 16
                rowbuf[r, pl.ds(0, 16)] = a_b[fr, pl.ds(fc, 16)]
                return 0
            lax.fori_loop(0, EBA, row, 0)
            pltpu.sync_copy(rowbuf, s_acc.at[dst_v.at[j]], add=True)
            return 0

        lax.fori_loop(0, ABATCHA, batch, 0)
        plsc.subcore_barrier()

        @pl.when(cid == 0)
        def _():
            _writeback(s_acc, sp0_hbm, tid)

        @pl.when(cid == 1)
        def _():
            _writeback(s_acc, sp1_hbm, tid)

    return k


@functools.lru_cache(maxsize=None)
def _make_attn_norm(h):
    @functools.partial(
        pl.kernel,
        out_type=jax.ShapeDtypeStruct((FLA, 128), jnp.float32),
        mesh=_mesh(),
        scratch_types=[
            pltpu.VMEM((ABATCHA, EBA), jnp.int32),
            pltpu.VMEM((8, 128), jnp.float32),
            pltpu.VMEM((8, 128), jnp.float32),
            pltpu.VMEM((EBA, 128), jnp.float32),
            pltpu.VMEM((EBA, 128), jnp.float32),
        ],
    )
    def k(dstA_hbm, a_hbm, sp0_hbm, sp1_hbm, alpha_hbm,
          dst_v, a_b, arow, s0_r, s1_r):
        cid = lax.axis_index("c")
        tid = lax.axis_index("s")
        w = cid * 16 + tid

        pltpu.sync_copy(dstA_hbm.at[w], dst_v)

        def batch(j, _):
            pltpu.sync_copy(sp0_hbm.at[dst_v.at[j]], s0_r)
            pltpu.sync_copy(sp1_hbm.at[dst_v.at[j]], s1_r)
            fbase = pl.multiple_of((w * ABATCHA + j) * 8, 8)
            pltpu.sync_copy(a_hbm.at[pl.ds(fbase, 8)], a_b)

            def row(r, _):
                fr = r // 8
                fc = (r % 8) * 16
                sv = s0_r[r, pl.ds(0, 16)] + s1_r[r, pl.ds(0, 16)]
                arow[fr, pl.ds(fc, 16)] = a_b[fr, pl.ds(fc, 16)] / sv
                return 0
            lax.fori_loop(0, EBA, row, 0)
            pltpu.sync_copy(arow, alpha_hbm.at[pl.ds(fbase, 8)])
            return 0

        lax.fori_loop(0, ABATCHA, batch, 0)

    del h
    return k


@functools.lru_cache(maxsize=None)
def _make_msg(ha, hb):
    @functools.partial(
        pl.kernel,
        out_type=[
            jax.ShapeDtypeStruct((N, 128), jnp.float32),
            jax.ShapeDtypeStruct((N, 128), jnp.float32),
        ],
        mesh=_mesh(),
        scratch_types=[
            pltpu.VMEM((MBATCH, EB), jnp.int32),
            pltpu.VMEM((MBATCH, EB), jnp.int32),
            pltpu.VMEM((16, 128), jnp.float32),
            pltpu.VMEM((EB, 128), jnp.float32),
            pltpu.VMEM_SHARED((NROWP, 128), jnp.float32),
        ],
    )
    def k(srcM_hbm, dstM_hbm, alpha_hbm, tbl_a_hbm, tbl_b_hbm,
          out_a_hbm, out_b_hbm, src_v, dst_v, alpha_b, rows, acc):
        cid = lax.axis_index("c")
        tid = lax.axis_index("s")

        pltpu.sync_copy(srcM_hbm.at[tid], src_v)
        pltpu.sync_copy(dstM_hbm.at[tid], dst_v)

        def zrow(r, _):
            for kk in range(8):
                rows[r, pl.ds(kk * 16, 16)] = jnp.zeros((16,), jnp.float32)
            return 0
        lax.fori_loop(0, EB, zrow, 0)
        for i in range(STRIPE // EB):
            base = pl.multiple_of(tid * STRIPE + i * EB, 8)
            pltpu.sync_copy(rows, acc.at[pl.ds(base, EB)])
        plsc.subcore_barrier()

        def run(head, tbl_hbm, out_hbm):
            def batch(j, _):
                fbase = pl.multiple_of((tid * MBATCH + j) * 16, 16)
                pltpu.sync_copy(alpha_hbm.at[pl.ds(fbase, 16)], alpha_b)
                pltpu.sync_copy(tbl_hbm.at[src_v.at[j]], rows)

                def scale(g, _):
                    for rr in range(16):
                        a16 = alpha_b[2 * g + rr // 8,
                                      pl.ds((rr % 8) * 16, 16)]
                        asc = a16[head]
                        r = g * 16 + rr
                        for kk in range(8):
                            v = rows[r, pl.ds(kk * 16, 16)]
                            rows[r, pl.ds(kk * 16, 16)] = v * asc
                    return 0
                lax.fori_loop(0, EB // 16, scale, 0)
                pltpu.sync_copy(rows, acc.at[dst_v.at[j]], add=True)
                return 0

            lax.fori_loop(0, MBATCH, batch, 0)
            plsc.subcore_barrier()
            _writeback(acc, out_hbm, tid)

        @pl.when(cid == 0)
        def _():
            run(ha, tbl_a_hbm, out_a_hbm)

        @pl.when(cid == 1)
        def _():
            run(hb, tbl_b_hbm, out_b_hbm)

    return k


@functools.lru_cache(maxsize=None)
def _make_msg_split():
    @functools.partial(
        pl.kernel,
        out_type=[
            jax.ShapeDtypeStruct((N, 128), jnp.float32),
            jax.ShapeDtypeStruct((N, 128), jnp.float32),
        ],
        mesh=_mesh(),
        scratch_types=[
            pltpu.VMEM((ABATCHA, EBA), jnp.int32),
            pltpu.VMEM((ABATCHA, EBA), jnp.int32),
            pltpu.VMEM((8, 128), jnp.float32),
            pltpu.VMEM((EBA, 128), jnp.float32),
            pltpu.VMEM_SHARED((NROWP, 128), jnp.float32),
        ],
    )
    def k(srcA_hbm, dstA_hbm, alpha_hbm, tbl_hbm, p0_hbm, p1_hbm,
          src_v, dst_v, alpha_b, rows, acc):
        cid = lax.axis_index("c")
        tid = lax.axis_index("s")
        w = cid * 16 + tid

        pltpu.sync_copy(srcA_hbm.at[w], src_v)
        pltpu.sync_copy(dstA_hbm.at[w], dst_v)

        def zrow(r, _):
            for kk in range(8):
                rows[r, pl.ds(kk * 16, 16)] = jnp.zeros((16,), jnp.float32)
            return 0
        lax.fori_loop(0, EBA, zrow, 0)
        for i in range(STRIPE // EBA):
            base = pl.multiple_of(tid * STRIPE + i * EBA, 8)
            pltpu.sync_copy(rows, acc.at[pl.ds(base, EBA)])
        plsc.subcore_barrier()

        def batch(j, _):
            fbase = pl.multiple_of((w * ABATCHA + j) * 8, 8)
            pltpu.sync_copy(alpha_hbm.at[pl.ds(fbase, 8)], alpha_b)
            pltpu.sync_copy(tbl_hbm.at[src_v.at[j]], rows)

            def scale(g, _):
                for rr in range(16):
                    a16 = alpha_b[2 * g + rr // 8, pl.ds((rr % 8) * 16, 16)]
                    asc = a16[0]
                    r = g * 16 + rr
                    for kk in range(8):
                        v = rows[r, pl.ds(kk * 16, 16)]
                        rows[r, pl.ds(kk * 16, 16)] = v * asc
                return 0
            lax.fori_loop(0, EBA // 16, scale, 0)
            pltpu.sync_copy(rows, acc.at[dst_v.at[j]], add=True)
            return 0

        lax.fori_loop(0, ABATCHA, batch, 0)
        plsc.subcore_barrier()

        @pl.when(cid == 0)
        def _():
            _writeback(acc, p0_hbm, tid)

        @pl.when(cid == 1)
        def _():
            _writeback(acc, p1_hbm, tid)

    return k


def kernel(x, edge_index, edge_attr, W0, as0, ad0, We0, ae0, b0,
           W1, as1, ad1, We1, ae1, b1, L1, bL1, L2, bL2, L3, bL3):
    f32 = jnp.float32
    loop = jnp.arange(N, dtype=edge_index.dtype)
    padz = jnp.zeros((EPAD - ET,), edge_index.dtype)
    src = jnp.concatenate([edge_index[0], loop, padz])
    dst = jnp.concatenate([edge_index[1], loop, padz])
    srcA = src.reshape(WATT, ABATCHA, EBA)
    dstA = dst.reshape(WATT, ABATCHA, EBA)
    srcM = src.reshape(16, MBATCH, EB)
    dstM = dst.reshape(16, MBATCH, EB)

    vs0 = jnp.einsum('khc,hc->kh', W0.reshape(DIN, H0, HID), as0[0])
    vd0 = jnp.einsum('khc,hc->kh', W0.reshape(DIN, H0, HID), ad0[0])
    VS0 = jnp.concatenate([vs0, vd0], axis=1)
    ue0 = jnp.einsum('dhc,hc->dh', We0.reshape(DE, H0, HID), ae0[0])
    ue1 = We1 @ ae1[0, 0]
    UEp = jnp.concatenate(
        [ue0, ue1[:, None], jnp.zeros((DE, 11), f32)], axis=1)
    vs1 = W1 @ as1[0, 0]
    vd1 = W1 @ ad1[0, 0]
    VS1r = jnp.concatenate(
        [vs1[:, None], vd1[:, None], jnp.zeros((D0, 6), f32)],
        axis=1).reshape(NCH0, 128, 8)
    W1r = W1.reshape(NCH0, 128, EMB)

    ale_cat, esum = _edge_proj(edge_attr, UEp)
    mean_e = esum[0] / E
    ale_self = mean_e @ UEp
    zpad = jnp.zeros((EPAD - ET, 16), f32)
    ale_full = jnp.concatenate(
        [ale_cat, jnp.broadcast_to(ale_self, (N, 16)), zpad])
    ale0 = jnp.concatenate(
        [ale_full[:, 0:4], jnp.zeros((EPAD, 12), f32)], 1).reshape(FLA, 128)
    ale1 = jnp.concatenate(
        [ale_full[:, 4:5], jnp.zeros((EPAD, 15), f32)], 1).reshape(FLA, 128)

    h0c, alsd0 = _layer0_mm(x, W0, VS0)
    tabS0 = jnp.concatenate([alsd0[:, 0:4], jnp.zeros((N, 124), f32)], 1)
    tabD0 = jnp.concatenate([alsd0[:, 4:8], jnp.zeros((N, 124), f32)], 1)
    al0 = _make_attn_logits(4)(srcA, dstA, tabS0, tabD0, ale0)
    a0 = _exp_flat(al0)
    sp0a, sp0b = _make_attn_s()(dstA, a0)
    alpha0 = _make_attn_norm(4)(dstA, a0, sp0a, sp0b)
    out_c = [None] * NCH0
    for pair in range(4):
        ca, cb = 2 * pair, 2 * pair + 1
        out_c[ca], out_c[cb] = _make_msg(ca // 2, cb // 2)(
            srcM, dstM, alpha0, h0c[ca], h0c[cb])
    out0 = jnp.stack(out_c)

    stats = _bn_stats(out0)
    h1, alsd1 = _layer1_mm(out0, stats, W1r, VS1r)

    tabS1 = jnp.concatenate([alsd1[:, 0:1], jnp.zeros((N, 127), f32)], 1)
    tabD1 = jnp.concatenate([alsd1[:, 1:2], jnp.zeros((N, 127), f32)], 1)
    al1 = _make_attn_logits(1)(srcA, dstA, tabS1, tabD1, ale1)
    a1 = _exp_flat(al1)
    sp1a, sp1b = _make_attn_s()(dstA, a1)
    alpha1 = _make_attn_norm(1)(dstA, a1, sp1a, sp1b)
    p0, p1 = _make_msg_split()(srcA, dstA, alpha1, h1)

    emb, z1, st1 = _decoder_a(p0, p1, b1.reshape(1, EMB),
                              L1, bL1.reshape(1, 2 * HID))
    recon = _decoder_b(z1, st1, L2, bL2.reshape(1, HID),
                       L3, bL3.reshape(1, DIN))
    return (emb, recon)


# speedup vs baseline: 1.0860x; 1.0860x over previous
import functools

import jax
import jax.numpy as jnp
from jax import lax
from jax.experimental import pallas as pl
from jax.experimental.pallas import tpu as pltpu
from jax.experimental.pallas import tpu_sc as plsc

N = 10000
E = 160000
ET = 170000
EPAD = 172032
DIN = 256
DE = 16
H0 = 4
HID = 256
D0 = 1024
EMB = 128
NCH0 = 8
BN_EPS = 1e-5

NB = 10
BNODE = 1000
EB = 128
WATT = 32
ABATCH = EPAD // (WATT * EB)
EBA = 64
ABATCHA = EPAD // (WATT * EBA)
FLA = EPAD * 16 // 128
MBATCH = EPAD // (16 * EB)
EBM = 64
MBATCH2 = EPAD // (16 * EBM)
NROWP = 10240
STRIPE = 640


def _edge_proj(edge_attr, UEp):
    def body(ea_ref, ue_ref, ale_ref, es_ref):
        i = pl.program_id(0)
        blk = ea_ref[...]
        ale_ref[...] = jnp.dot(blk, ue_ref[...],
                               preferred_element_type=jnp.float32)

        @pl.when(i == 0)
        def _():
            es_ref[...] = jnp.zeros_like(es_ref)
        es_ref[...] += jnp.broadcast_to(
            jnp.sum(blk, axis=0, keepdims=True), es_ref.shape)

    return pl.pallas_call(
        body,
        grid=(20,),
        in_specs=[
            pl.BlockSpec((8000, DE), lambda i: (i, 0)),
            pl.BlockSpec((DE, 16), lambda i: (0, 0)),
        ],
        out_specs=[
            pl.BlockSpec((8000, 16), lambda i: (i, 0)),
            pl.BlockSpec((8, DE), lambda i: (0, 0)),
        ],
        out_shape=[
            jax.ShapeDtypeStruct((E, 16), jnp.float32),
            jax.ShapeDtypeStruct((8, DE), jnp.float32),
        ],
    )(edge_attr, UEp)


def _layer0_mm(x, W0, VS0):
    def body(x_ref, w_ref, vs_ref, h_ref, al_ref):
        c = pl.program_id(1)
        xb = x_ref[...]
        h_ref[0] = jnp.dot(xb, w_ref[...], preferred_element_type=jnp.float32)

        @pl.when(c == 0)
        def _():
            al_ref[...] = jnp.dot(xb, vs_ref[...],
                                  preferred_element_type=jnp.float32)

    return pl.pallas_call(
        body,
        grid=(NB, NCH0),
        in_specs=[
            pl.BlockSpec((BNODE, DIN), lambda i, c: (i, 0)),
            pl.BlockSpec((DIN, 128), lambda i, c: (0, c)),
            pl.BlockSpec((DIN, 8), lambda i, c: (0, 0)),
        ],
        out_specs=[
            pl.BlockSpec((1, BNODE, 128), lambda i, c: (c, i, 0)),
            pl.BlockSpec((BNODE, 8), lambda i, c: (i, 0)),
        ],
        out_shape=[
            jax.ShapeDtypeStruct((NCH0, N, 128), jnp.float32),
            jax.ShapeDtypeStruct((N, 8), jnp.float32),
        ],
    )(x, W0, VS0)


def _bn_stats(h0c):
    def body(h_ref, st_ref):
        i = pl.program_id(1)

        @pl.when(i == 0)
        def _():
            st_ref[...] = jnp.zeros_like(st_ref)
        xb = h_ref[0]
        st_ref[:, 0:1, :] += jnp.sum(xb, axis=0, keepdims=True)[None]
        st_ref[:, 1:2, :] += jnp.sum(xb * xb, axis=0, keepdims=True)[None]

    return pl.pallas_call(
        body,
        grid=(NCH0, NB),
        in_specs=[pl.BlockSpec((1, BNODE, 128), lambda c, i: (c, i, 0))],
        out_specs=pl.BlockSpec((1, 2, 128), lambda c, i: (c, 0, 0)),
        out_shape=jax.ShapeDtypeStruct((NCH0, 2, 128), jnp.float32),
    )(h0c)


def _layer1_mm(h0c, stats, W1r, VS1r):
    def body(h_ref, st_ref, w_ref, vs_ref, h1_ref, al_ref):
        k = pl.program_id(1)
        m = st_ref[0, 0:1, :] / N
        ex2 = st_ref[0, 1:2, :] / N
        inv = 1.0 / jnp.sqrt(ex2 - m * m + BN_EPS)
        xn = (h_ref[0] - m) * inv
        he = jnp.where(xn > 0, xn, jnp.exp(xn) - 1.0)

        @pl.when(k == 0)
        def _():
            h1_ref[...] = jnp.zeros_like(h1_ref)
            al_ref[...] = jnp.zeros_like(al_ref)
        h1_ref[...] += jnp.dot(he, w_ref[0],
                               preferred_element_type=jnp.float32)
        al_ref[...] += jnp.dot(he, vs_ref[0],
                               preferred_element_type=jnp.float32)

    return pl.pallas_call(
        body,
        grid=(NB, NCH0),
        in_specs=[
            pl.BlockSpec((1, BNODE, 128), lambda i, k: (k, i, 0)),
            pl.BlockSpec((1, 2, 128), lambda i, k: (k, 0, 0)),
            pl.BlockSpec((1, 128, EMB), lambda i, k: (k, 0, 0)),
            pl.BlockSpec((1, 128, 8), lambda i, k: (k, 0, 0)),
        ],
        out_specs=[
            pl.BlockSpec((BNODE, EMB), lambda i, k: (i, 0)),
            pl.BlockSpec((BNODE, 8), lambda i, k: (i, 0)),
        ],
        out_shape=[
            jax.ShapeDtypeStruct((N, EMB), jnp.float32),
            jax.ShapeDtypeStruct((N, 8), jnp.float32),
        ],
    )(h0c, stats, W1r, VS1r)


def _decoder_a(p0, p1, b1, L1, bL1):
    def body(x_ref, y_ref, b1_ref, l1_ref, bl1_ref, emb_ref, z1_ref, st_ref):
        i = pl.program_id(0)
        embb = x_ref[...] + y_ref[...] + b1_ref[...]
        emb_ref[...] = embb
        z = jnp.dot(embb, l1_ref[...], preferred_element_type=jnp.float32)
        z = jnp.maximum(z + bl1_ref[...], 0.0)
        z1_ref[...] = z

        @pl.when(i == 0)
        def _():
            st_ref[...] = jnp.zeros_like(st_ref)
        st_ref[:, 0:1, :] += jnp.sum(z, axis=0, keepdims=True)[None]
        st_ref[:, 1:2, :] += jnp.sum(z * z, axis=0, keepdims=True)[None]

    return pl.pallas_call(
        body,
        grid=(NB,),
        in_specs=[
            pl.BlockSpec((BNODE, EMB), lambda i: (i, 0)),
            pl.BlockSpec((BNODE, EMB), lambda i: (i, 0)),
            pl.BlockSpec((1, EMB), lambda i: (0, 0)),
            pl.BlockSpec((EMB, 2 * HID), lambda i: (0, 0)),
            pl.BlockSpec((1, 2 * HID), lambda i: (0, 0)),
        ],
        out_specs=[
            pl.BlockSpec((BNODE, EMB), lambda i: (i, 0)),
            pl.BlockSpec((BNODE, 2 * HID), lambda i: (i, 0)),
            pl.BlockSpec((1, 2, 2 * HID), lambda i: (0, 0, 0)),
        ],
        out_shape=[
            jax.ShapeDtypeStruct((N, EMB), jnp.float32),
            jax.ShapeDtypeStruct((N, 2 * HID), jnp.float32),
            jax.ShapeDtypeStruct((1, 2, 2 * HID), jnp.float32),
        ],
    )(p0, p1, b1, L1, bL1)


def _decoder_b(z1, st1, L2, bL2, L3, bL3):
    def body(z_ref, st_ref, l2_ref, bl2_ref, l3_ref, bl3_ref, out_ref):
        m = st_ref[0, 0:1, :] / N
        ex2 = st_ref[0, 1:2, :] / N
        inv = 1.0 / jnp.sqrt(ex2 - m * m + BN_EPS)
        zn = (z_ref[...] - m) * inv
        z2 = jnp.dot(zn, l2_ref[...], preferred_element_type=jnp.float32)
        z2 = jnp.maximum(z2 + bl2_ref[...], 0.0)
        r = jnp.dot(z2, l3_ref[...], preferred_element_type=jnp.float32)
        out_ref[...] = r + bl3_ref[...]

    return pl.pallas_call(
        body,
        grid=(NB,),
        in_specs=[
            pl.BlockSpec((BNODE, 2 * HID), lambda i: (i, 0)),
            pl.BlockSpec((1, 2, 2 * HID), lambda i: (0, 0, 0)),
            pl.BlockSpec((2 * HID, HID), lambda i: (0, 0)),
            pl.BlockSpec((1, HID), lambda i: (0, 0)),
            pl.BlockSpec((HID, DIN), lambda i: (0, 0)),
            pl.BlockSpec((1, DIN), lambda i: (0, 0)),
        ],
        out_specs=pl.BlockSpec((BNODE, DIN), lambda i: (i, 0)),
        out_shape=jax.ShapeDtypeStruct((N, DIN), jnp.float32),
    )(z1, st1, L2, bL2, L3, bL3)


def _exp_flat(al):
    def body(al_ref, a_ref):
        a_ref[...] = jnp.exp(al_ref[...])

    return pl.pallas_call(
        body,
        grid=(8,),
        in_specs=[pl.BlockSpec((FLA // 8, 128), lambda i: (i, 0))],
        out_specs=pl.BlockSpec((FLA // 8, 128), lambda i: (i, 0)),
        out_shape=jax.ShapeDtypeStruct((FLA, 128), jnp.float32),
    )(al)


@functools.lru_cache(maxsize=None)
def _mesh():
    return plsc.VectorSubcoreMesh(core_axis_name="c", subcore_axis_name="s")


def _zero_acc(acc, zbuf, tid):
    nk = zbuf.shape[1] // 16

    def zrow(r, _):
        for kk in range(nk):
            zbuf[r, pl.ds(kk * 16, 16)] = jnp.zeros((16,), jnp.float32)
        return 0
    lax.fori_loop(0, 128, zrow, 0)
    for i in range(STRIPE // 128):
        base = pl.multiple_of(tid * STRIPE + i * 128, 128)
        pltpu.sync_copy(zbuf, acc.at[pl.ds(base, 128)])


def _writeback(acc, out_hbm, tid):
    base = pl.multiple_of(tid * STRIPE, 8)

    @pl.when(tid < 15)
    def _():
        pltpu.sync_copy(acc.at[pl.ds(base, STRIPE)],
                        out_hbm.at[pl.ds(base, STRIPE)])

    @pl.when(tid == 15)
    def _():
        pltpu.sync_copy(acc.at[pl.ds(base, N - 15 * STRIPE)],
                        out_hbm.at[pl.ds(base, N - 15 * STRIPE)])


@functools.lru_cache(maxsize=None)
def _make_attn_logits(h):
    @functools.partial(
        pl.kernel,
        out_type=jax.ShapeDtypeStruct((FLA, 128), jnp.float32),
        mesh=_mesh(),
        scratch_types=[
            pltpu.VMEM((ABATCHA, EBA), jnp.int32),
            pltpu.VMEM((ABATCHA, EBA), jnp.int32),
            pltpu.VMEM((EBA, 128), jnp.float32),
            pltpu.VMEM((EBA, 128), jnp.float32),
            pltpu.VMEM((8, 128), jnp.float32),
            pltpu.VMEM((8, 128), jnp.float32),
        ],
    )
    def k(srcA_hbm, dstA_hbm, tabS_hbm, tabD_hbm, ale_hbm, al_hbm,
          src_v, dst_v, als_r, ald_r, ale_b, a_b):
        cid = lax.axis_index("c")
        tid = lax.axis_index("s")
        w = cid * 16 + tid

        pltpu.sync_copy(srcA_hbm.at[w], src_v)
        pltpu.sync_copy(dstA_hbm.at[w], dst_v)

        def batch(j, _):
            pltpu.sync_copy(tabS_hbm.at[src_v.at[j]], als_r)
            pltpu.sync_copy(tabD_hbm.at[dst_v.at[j]], ald_r)
            fbase = pl.multiple_of((w * ABATCHA + j) * 8, 8)
            pltpu.sync_copy(ale_hbm.at[pl.ds(fbase, 8)], ale_b)

            def row(r, _):
                fr = r // 8
                fc = (r % 8) * 16
                al = (als_r[r, pl.ds(0, 16)] + ald_r[r, pl.ds(0, 16)]
                      + ale_b[fr, pl.ds(fc, 16)])
                al = jnp.where(al > 0, al, al * jnp.float32(0.2))
                valid = (w * ABATCHA + j) * EBA + r < ET

                @pl.when(valid)
                def _():
                    a_b[fr, pl.ds(fc, 16)] = al

                @pl.when(jnp.logical_not(valid))
                def _():
                    a_b[fr, pl.ds(fc, 16)] = jnp.full(
                        (16,), -1e30, jnp.float32)
                return 0
            lax.fori_loop(0, EBA, row, 0)
            pltpu.sync_copy(a_b, al_hbm.at[pl.ds(fbase, 8)])
            return 0

        lax.fori_loop(0, ABATCHA, batch, 0)

    del h
    return k


@functools.lru_cache(maxsize=None)
def _make_attn_s():
    @functools.partial(
        pl.kernel,
        out_type=[
            jax.ShapeDtypeStruct((N, 128), jnp.float32),
            jax.ShapeDtypeStruct((N, 128), jnp.float32),
        ],
        mesh=_mesh(),
        scratch_types=[
            pltpu.VMEM((ABATCHA, EBA), jnp.int32),
            pltpu.VMEM((8, 128), jnp.float32),
            pltpu.VMEM((EBA, 128), jnp.float32),
            pltpu.VMEM_SHARED((NROWP, 128), jnp.float32),
        ],
    )
    def k(dstA_hbm, a_hbm, sp0_hbm, sp1_hbm, dst_v, a_b, rowbuf, s_acc):
        cid = lax.axis_index("c")
        tid = lax.axis_index("s")
        w = cid * 16 + tid

        pltpu.sync_copy(dstA_hbm.at[w], dst_v)

        def zrow(r, _):
            for kk in range(8):
                rowbuf[r, pl.ds(kk * 16, 16)] = jnp.zeros((16,), jnp.float32)
            return 0
        lax.fori_loop(0, EBA, zrow, 0)
        for i in range(STRIPE // EBA):
            base = pl.multiple_of(tid * STRIPE + i * EBA, 8)
            pltpu.sync_copy(rowbuf, s_acc.at[pl.ds(base, EBA)])
        plsc.subcore_barrier()

        def batch(j, _):
            fbase = pl.multiple_of((w * ABATCHA + j) * 8, 8)
            pltpu.sync_copy(a_hbm.at[pl.ds(fbase, 8)], a_b)

            def row(r, _):
                fr = r // 8
                fc = (r % 8) * 16
                rowbuf[r, pl.ds(0, 16)] = a_b[fr, pl.ds(fc, 16)]
                return 0
            lax.fori_loop(0, EBA, row, 0)
            pltpu.sync_copy(rowbuf, s_acc.at[dst_v.at[j]], add=True)
            return 0

        lax.fori_loop(0, ABATCHA, batch, 0)
        plsc.subcore_barrier()

        @pl.when(cid == 0)
        def _():
            _writeback(s_acc, sp0_hbm, tid)

        @pl.when(cid == 1)
        def _():
            _writeback(s_acc, sp1_hbm, tid)

    return k


@functools.lru_cache(maxsize=None)
def _make_attn_norm(h):
    @functools.partial(
        pl.kernel,
        out_type=jax.ShapeDtypeStruct((FLA, 128), jnp.float32),
        mesh=_mesh(),
        scratch_types=[
            pltpu.VMEM((ABATCHA, EBA), jnp.int32),
            pltpu.VMEM((8, 128), jnp.float32),
            pltpu.VMEM((8, 128), jnp.float32),
            pltpu.VMEM((EBA, 128), jnp.float32),
            pltpu.VMEM((EBA, 128), jnp.float32),
        ],
    )
    def k(dstA_hbm, a_hbm, sp0_hbm, sp1_hbm, alpha_hbm,
          dst_v, a_b, arow, s0_r, s1_r):
        cid = lax.axis_index("c")
        tid = lax.axis_index("s")
        w = cid * 16 + tid

        pltpu.sync_copy(dstA_hbm.at[w], dst_v)

        def batch(j, _):
            pltpu.sync_copy(sp0_hbm.at[dst_v.at[j]], s0_r)
            pltpu.sync_copy(sp1_hbm.at[dst_v.at[j]], s1_r)
            fbase = pl.multiple_of((w * ABATCHA + j) * 8, 8)
            pltpu.sync_copy(a_hbm.at[pl.ds(fbase, 8)], a_b)

            def row(r, _):
                fr = r // 8
                fc = (r % 8) * 16
                sv = s0_r[r, pl.ds(0, 16)] + s1_r[r, pl.ds(0, 16)]
                arow[fr, pl.ds(fc, 16)] = a_b[fr, pl.ds(fc, 16)] / sv
                return 0
            lax.fori_loop(0, EBA, row, 0)
            pltpu.sync_copy(arow, alpha_hbm.at[pl.ds(fbase, 8)])
            return 0

        lax.fori_loop(0, ABATCHA, batch, 0)

    del h
    return k


@functools.lru_cache(maxsize=None)
def _make_msg(ha, hb):
    @functools.partial(
        pl.kernel,
        out_type=[
            jax.ShapeDtypeStruct((N, 128), jnp.float32),
            jax.ShapeDtypeStruct((N, 128), jnp.float32),
        ],
        mesh=_mesh(),
        scratch_types=[
            pltpu.VMEM((MBATCH, EB), jnp.int32),
            pltpu.VMEM((EB,), jnp.int32),
            pltpu.VMEM((16, 128), jnp.float32),
            pltpu.VMEM((EB, 128), jnp.float32),
            pltpu.VMEM((EB, 128), jnp.float32),
            pltpu.SemaphoreType.DMA,
            pltpu.SemaphoreType.DMA,
            pltpu.VMEM_SHARED((NROWP, 128), jnp.float32),
        ],
    )
    def k(srcM_hbm, dstM_hbm, alpha_hbm, tbl_a_hbm, tbl_b_hbm,
          out_a_hbm, out_b_hbm, src_v, dst_v, alpha_b, rows, rows2,
          sem0, sem1, acc):
        cid = lax.axis_index("c")
        tid = lax.axis_index("s")

        pltpu.sync_copy(srcM_hbm.at[tid], src_v)

        def zrow(r, _):
            for kk in range(8):
                rows[r, pl.ds(kk * 16, 16)] = jnp.zeros((16,), jnp.float32)
            return 0
        lax.fori_loop(0, EB, zrow, 0)
        for i in range(STRIPE // EB):
            base = pl.multiple_of(tid * STRIPE + i * EB, 8)
            pltpu.sync_copy(rows, acc.at[pl.ds(base, EB)])
        plsc.subcore_barrier()

        def run(head, tbl_hbm, out_hbm):
            def do_half(j, buf, sem):
                fbase = pl.multiple_of((tid * MBATCH + j) * 16, 16)
                pltpu.sync_copy(alpha_hbm.at[pl.ds(fbase, 16)], alpha_b)
                pltpu.sync_copy(dstM_hbm.at[tid, j], dst_v)
                pltpu.make_async_copy(tbl_hbm.at[src_v.at[j]], buf,
                                      sem).wait()

                def scale(g, _):
                    for rr in range(16):
                        a16 = alpha_b[2 * g + rr // 8,
                                      pl.ds((rr % 8) * 16, 16)]
                        asc = a16[head]
                        r = g * 16 + rr
                        for kk in range(8):
                            v = buf[r, pl.ds(kk * 16, 16)]
                            buf[r, pl.ds(kk * 16, 16)] = v * asc
                    return 0
                lax.fori_loop(0, EB // 16, scale, 0)
                pltpu.sync_copy(buf, acc.at[dst_v], add=True)

            pltpu.async_copy(tbl_hbm.at[src_v.at[0]], rows, sem0)

            def pair(jp, _):
                j0 = 2 * jp
                pltpu.async_copy(tbl_hbm.at[src_v.at[j0 + 1]], rows2, sem1)
                do_half(j0, rows, sem0)

                @pl.when(jp + 1 < MBATCH // 2)
                def _():
                    pltpu.async_copy(tbl_hbm.at[src_v.at[j0 + 2]], rows,
                                     sem0)
                do_half(j0 + 1, rows2, sem1)
                return 0

            lax.fori_loop(0, MBATCH // 2, pair, 0)
            plsc.subcore_barrier()
            _writeback(acc, out_hbm, tid)

        @pl.when(cid == 0)
        def _():
            run(ha, tbl_a_hbm, out_a_hbm)

        @pl.when(cid == 1)
        def _():
            run(hb, tbl_b_hbm, out_b_hbm)

    return k


@functools.lru_cache(maxsize=None)
def _make_msg_split():
    @functools.partial(
        pl.kernel,
        out_type=[
            jax.ShapeDtypeStruct((N, 128), jnp.float32),
            jax.ShapeDtypeStruct((N, 128), jnp.float32),
        ],
        mesh=_mesh(),
        scratch_types=[
            pltpu.VMEM((ABATCHA, EBA), jnp.int32),
            pltpu.VMEM((ABATCHA, EBA), jnp.int32),
            pltpu.VMEM((8, 128), jnp.float32),
            pltpu.VMEM((EBA, 128), jnp.float32),
            pltpu.VMEM_SHARED((NROWP, 128), jnp.float32),
        ],
    )
    def k(srcA_hbm, dstA_hbm, alpha_hbm, tbl_hbm, p0_hbm, p1_hbm,
          src_v, dst_v, alpha_b, rows, acc):
        cid = lax.axis_index("c")
        tid = lax.axis_index("s")
        w = cid * 16 + tid

        pltpu.sync_copy(srcA_hbm.at[w], src_v)
        pltpu.sync_copy(dstA_hbm.at[w], dst_v)

        def zrow(r, _):
            for kk in range(8):
                rows[r, pl.ds(kk * 16, 16)] = jnp.zeros((16,), jnp.float32)
            return 0
        lax.fori_loop(0, EBA, zrow, 0)
        for i in range(STRIPE // EBA):
            base = pl.multiple_of(tid * STRIPE + i * EBA, 8)
            pltpu.sync_copy(rows, acc.at[pl.ds(base, EBA)])
        plsc.subcore_barrier()

        def batch(j, _):
            fbase = pl.multiple_of((w * ABATCHA + j) * 8, 8)
            pltpu.sync_copy(alpha_hbm.at[pl.ds(fbase, 8)], alpha_b)
            pltpu.sync_copy(tbl_hbm.at[src_v.at[j]], rows)

            def scale(g, _):
                for rr in range(16):
                    a16 = alpha_b[2 * g + rr // 8, pl.ds((rr % 8) * 16, 16)]
                    asc = a16[0]
                    r = g * 16 + rr
                    for kk in range(8):
                        v = rows[r, pl.ds(kk * 16, 16)]
                        rows[r, pl.ds(kk * 16, 16)] = v * asc
                return 0
            lax.fori_loop(0, EBA // 16, scale, 0)
            pltpu.sync_copy(rows, acc.at[dst_v.at[j]], add=True)
            return 0

        lax.fori_loop(0, ABATCHA, batch, 0)
        plsc.subcore_barrier()

        @pl.when(cid == 0)
        def _():
            _writeback(acc, p0_hbm, tid)

        @pl.when(cid == 1)
        def _():
            _writeback(acc, p1_hbm, tid)

    return k


def kernel(x, edge_index, edge_attr, W0, as0, ad0, We0, ae0, b0,
           W1, as1, ad1, We1, ae1, b1, L1, bL1, L2, bL2, L3, bL3):
    f32 = jnp.float32
    loop = jnp.arange(N, dtype=edge_index.dtype)
    padz = jnp.zeros((EPAD - ET,), edge_index.dtype)
    src = jnp.concatenate([edge_index[0], loop, padz])
    dst = jnp.concatenate([edge_index[1], loop, padz])
    srcA = src.reshape(WATT, ABATCHA, EBA)
    dstA = dst.reshape(WATT, ABATCHA, EBA)
    srcM = src.reshape(16, MBATCH, EB)
    dstM = dst.reshape(16, MBATCH, EB)

    vs0 = jnp.einsum('khc,hc->kh', W0.reshape(DIN, H0, HID), as0[0])
    vd0 = jnp.einsum('khc,hc->kh', W0.reshape(DIN, H0, HID), ad0[0])
    VS0 = jnp.concatenate([vs0, vd0], axis=1)
    ue0 = jnp.einsum('dhc,hc->dh', We0.reshape(DE, H0, HID), ae0[0])
    ue1 = We1 @ ae1[0, 0]
    UEp = jnp.concatenate(
        [ue0, ue1[:, None], jnp.zeros((DE, 11), f32)], axis=1)
    vs1 = W1 @ as1[0, 0]
    vd1 = W1 @ ad1[0, 0]
    VS1r = jnp.concatenate(
        [vs1[:, None], vd1[:, None], jnp.zeros((D0, 6), f32)],
        axis=1).reshape(NCH0, 128, 8)
    W1r = W1.reshape(NCH0, 128, EMB)

    ale_cat, esum = _edge_proj(edge_attr, UEp)
    mean_e = esum[0] / E
    ale_self = mean_e @ UEp
    zpad = jnp.zeros((EPAD - ET, 16), f32)
    ale_full = jnp.concatenate(
        [ale_cat, jnp.broadcast_to(ale_self, (N, 16)), zpad])
    ale0 = jnp.concatenate(
        [ale_full[:, 0:4], jnp.zeros((EPAD, 12), f32)], 1).reshape(FLA, 128)
    ale1 = jnp.concatenate(
        [ale_full[:, 4:5], jnp.zeros((EPAD, 15), f32)], 1).reshape(FLA, 128)

    h0c, alsd0 = _layer0_mm(x, W0, VS0)
    tabS0 = jnp.concatenate([alsd0[:, 0:4], jnp.zeros((N, 124), f32)], 1)
    tabD0 = jnp.concatenate([alsd0[:, 4:8], jnp.zeros((N, 124), f32)], 1)
    al0 = _make_attn_logits(4)(srcA, dstA, tabS0, tabD0, ale0)
    a0 = _exp_flat(al0)
    sp0a, sp0b = _make_attn_s()(dstA, a0)
    alpha0 = _make_attn_norm(4)(dstA, a0, sp0a, sp0b)
    out_c = [None] * NCH0
    for pair in range(4):
        ca, cb = 2 * pair, 2 * pair + 1
        out_c[ca], out_c[cb] = _make_msg(ca // 2, cb // 2)(
            srcM, dstM, alpha0, h0c[ca], h0c[cb])
    out0 = jnp.stack(out_c)

    stats = _bn_stats(out0)
    h1, alsd1 = _layer1_mm(out0, stats, W1r, VS1r)

    tabS1 = jnp.concatenate([alsd1[:, 0:1], jnp.zeros((N, 127), f32)], 1)
    tabD1 = jnp.concatenate([alsd1[:, 1:2], jnp.zeros((N, 127), f32)], 1)
    al1 = _make_attn_logits(1)(srcA, dstA, tabS1, tabD1, ale1)
    a1 = _exp_flat(al1)
    sp1a, sp1b = _make_attn_s()(dstA, a1)
    alpha1 = _make_attn_norm(1)(dstA, a1, sp1a, sp1b)
    p0, p1 = _make_msg_split()(srcA, dstA, alpha1, h1)

    emb, z1, st1 = _decoder_a(p0, p1, b1.reshape(1, EMB),
                              L1, bL1.reshape(1, 2 * HID))
    recon = _decoder_b(z1, st1, L2, bL2.reshape(1, HID),
                       L3, bL3.reshape(1, DIN))
    return (emb, recon)
